# baseline (speedup 1.0000x reference)
import functools
import jax
import jax.numpy as jnp
from jax import lax
from jax.experimental import pallas as pl
from jax.experimental.pallas import tpu as pltpu
from jax.experimental.pallas import tpu_sc as plsc

N, D = 100000, 128
L = 16
NC, NS = 2, 16
NW = NC * NS
CHUNK = 128
STEP = NW * CHUNK
ITERS = (N + STEP - 1) // STEP


def _sc_body(x_hbm, max_a_hbm, agg_n_hbm, his_x_hbm, his_m_hbm, inv_w_hbm,
             out_hbm, xb, hb, ab, nb, mb, wb, pb, qb):
    wid = lax.axis_index("s") * NC + lax.axis_index("c")

    def body(it, carry):
        base = it * STEP + wid * CHUNK
        base = jnp.minimum(base, N - CHUNK)
        pltpu.sync_copy(x_hbm.at[pl.ds(base, CHUNK)], xb)
        pltpu.sync_copy(his_x_hbm.at[pl.ds(base, CHUNK)], hb)
        pltpu.sync_copy(max_a_hbm.at[pl.ds(base, CHUNK)], ab)
        pltpu.sync_copy(agg_n_hbm.at[pl.ds(base, CHUNK)], nb)
        pltpu.sync_copy(his_m_hbm.at[pl.ds(base, CHUNK)], mb)
        pltpu.sync_copy(inv_w_hbm.at[pl.ds(base, CHUNK)], wb)

        neg_inf = jnp.float32(-jnp.inf)
        for g in range(CHUNK // L):
            sl = pl.ds(g * L, L)
            max_a = ab[sl]
            his_m = mb[sl]
            beta = jnp.clip(1.0 - wb[sl] * nb[sl], 0.0, 1.0)
            max_m = jnp.maximum(max_a, his_m)
            dp = his_m - max_m
            dq = max_a - max_m
            dp = jnp.where(dp != dp, neg_inf, dp)
            dq = jnp.where(dq != dq, neg_inf, dq)
            p = jnp.exp(dp) * beta
            q = jnp.exp(dq)
            t = jnp.maximum(p + q, 1.0)
            inv_t = 1.0 / t
            pb[sl] = p * inv_t
            qb[sl] = q * inv_t

        def dense(g, _):
            pv = pb[pl.ds(g * L, L)]
            qv = qb[pl.ds(g * L, L)]
            for j in range(L):
                r = g * L + j
                p = pv[j]
                q = qv[j]
                for c in range(D // L):
                    cs = pl.ds(c * L, L)
                    xb[r, cs] = hb[r, cs] * p + xb[r, cs] * q
            return 0

        lax.fori_loop(0, CHUNK // L, dense, 0)
        pltpu.sync_copy(xb, out_hbm.at[pl.ds(base, CHUNK)])
        return carry

    lax.fori_loop(0, ITERS, body, 0)


def sc_call(x, max_a, agg_n, his_x, his_m, inv_w):
    mesh = plsc.VectorSubcoreMesh(core_axis_name="c", subcore_axis_name="s")
    f = functools.partial(
        pl.kernel,
        out_type=jax.ShapeDtypeStruct((N, D), jnp.float32),
        mesh=mesh,
        scratch_types=[
            pltpu.VMEM((CHUNK, D), jnp.float32),
            pltpu.VMEM((CHUNK, D), jnp.float32),
            pltpu.VMEM((CHUNK,), jnp.float32),
            pltpu.VMEM((CHUNK,), jnp.float32),
            pltpu.VMEM((CHUNK,), jnp.float32),
            pltpu.VMEM((CHUNK,), jnp.float32),
            pltpu.VMEM((CHUNK,), jnp.float32),
            pltpu.VMEM((CHUNK,), jnp.float32),
        ],
    )(_sc_body)
    return f(x, max_a, agg_n, his_x, his_m, inv_w)


def kernel(x, max_a, agg_n, his_x, his_m, inv_w):
    return sc_call(x, max_a, agg_n, his_x, his_m, inv_w)


# speedup vs baseline: 1.1996x; 1.1996x over previous
import functools
import jax
import jax.numpy as jnp
from jax import lax
from jax.experimental import pallas as pl
from jax.experimental.pallas import tpu as pltpu
from jax.experimental.pallas import tpu_sc as plsc

N, D = 100000, 128
BLOCK = 16384

L = 16
NC, NS = 2, 16
NW = NC * NS
CHUNK = 128
STEP = NW * CHUNK
N_SC = 6 * STEP
N_TC = N - N_SC


def _emma_body(x_ref, max_a_ref, agg_n_ref, his_x_ref, his_m_ref, inv_w_ref,
               out_ref):
    max_a = max_a_ref[...]
    his_m = his_m_ref[...]
    beta = jnp.clip(1.0 - inv_w_ref[...] * agg_n_ref[...], 0.0, 1.0)
    max_m = jnp.maximum(max_a, his_m)
    neg_inf = jnp.float32(-jnp.inf)
    dp = his_m - max_m
    dq = max_a - max_m
    dp = jnp.where(jnp.isnan(dp), neg_inf, dp)
    dq = jnp.where(jnp.isnan(dq), neg_inf, dq)
    p = jnp.exp(dp) * beta
    q = jnp.exp(dq)
    t = jnp.maximum(p + q, 1.0)
    inv_t = 1.0 / t
    p2 = (p * inv_t)[:, None]
    q2 = (q * inv_t)[:, None]
    out_ref[...] = his_x_ref[...] * p2 + x_ref[...] * q2


def _tc_call(x, max_a, agg_n, his_x, his_m, inv_w):
    n = x.shape[0]
    grid = (n + BLOCK - 1) // BLOCK
    row_spec = pl.BlockSpec((BLOCK, D), lambda i: (i, 0))
    vec_spec = pl.BlockSpec((BLOCK,), lambda i: (i,))
    return pl.pallas_call(
        _emma_body,
        grid=(grid,),
        in_specs=[row_spec, vec_spec, vec_spec, row_spec, vec_spec, vec_spec],
        out_specs=row_spec,
        out_shape=jax.ShapeDtypeStruct((n, D), jnp.float32),
        compiler_params=pltpu.CompilerParams(
            dimension_semantics=("arbitrary",),
        ),
    )(x, max_a, agg_n, his_x, his_m, inv_w)


def _sc_body(n_rows, iters, x_hbm, max_a_hbm, agg_n_hbm, his_x_hbm, his_m_hbm,
             inv_w_hbm, out_hbm, xb, hb, ab, nb, mb, wb, pb, qb):
    wid = lax.axis_index("s") * NC + lax.axis_index("c")

    def body(it, carry):
        base = it * STEP + wid * CHUNK
        base = jnp.minimum(base, n_rows - CHUNK)
        pltpu.sync_copy(x_hbm.at[pl.ds(base, CHUNK)], xb)
        pltpu.sync_copy(his_x_hbm.at[pl.ds(base, CHUNK)], hb)
        pltpu.sync_copy(max_a_hbm.at[pl.ds(base, CHUNK)], ab)
        pltpu.sync_copy(agg_n_hbm.at[pl.ds(base, CHUNK)], nb)
        pltpu.sync_copy(his_m_hbm.at[pl.ds(base, CHUNK)], mb)
        pltpu.sync_copy(inv_w_hbm.at[pl.ds(base, CHUNK)], wb)

        neg_inf = jnp.float32(-jnp.inf)
        for g in range(CHUNK // L):
            sl = pl.ds(g * L, L)
            max_a = ab[sl]
            his_m = mb[sl]
            beta = jnp.clip(1.0 - wb[sl] * nb[sl], 0.0, 1.0)
            max_m = jnp.maximum(max_a, his_m)
            dp = his_m - max_m
            dq = max_a - max_m
            dp = jnp.where(dp != dp, neg_inf, dp)
            dq = jnp.where(dq != dq, neg_inf, dq)
            p = jnp.exp(dp) * beta
            q = jnp.exp(dq)
            t = jnp.maximum(p + q, 1.0)
            inv_t = 1.0 / t
            pb[sl] = p * inv_t
            qb[sl] = q * inv_t

        def dense(g, _):
            pv = pb[pl.ds(g * L, L)]
            qv = qb[pl.ds(g * L, L)]
            for j in range(L):
                r = g * L + j
                p = pv[j]
                q = qv[j]
                for c in range(D // L):
                    cs = pl.ds(c * L, L)
                    xb[r, cs] = hb[r, cs] * p + xb[r, cs] * q
            return 0

        lax.fori_loop(0, CHUNK // L, dense, 0)
        pltpu.sync_copy(xb, out_hbm.at[pl.ds(base, CHUNK)])
        return carry

    lax.fori_loop(0, iters, body, 0)


def _sc_call(x, max_a, agg_n, his_x, his_m, inv_w):
    n = x.shape[0]
    iters = (n + STEP - 1) // STEP
    mesh = plsc.VectorSubcoreMesh(core_axis_name="c", subcore_axis_name="s")
    f = functools.partial(
        pl.kernel,
        out_type=jax.ShapeDtypeStruct((n, D), jnp.float32),
        mesh=mesh,
        scratch_types=[
            pltpu.VMEM((CHUNK, D), jnp.float32),
            pltpu.VMEM((CHUNK, D), jnp.float32),
            pltpu.VMEM((CHUNK,), jnp.float32),
            pltpu.VMEM((CHUNK,), jnp.float32),
            pltpu.VMEM((CHUNK,), jnp.float32),
            pltpu.VMEM((CHUNK,), jnp.float32),
            pltpu.VMEM((CHUNK,), jnp.float32),
            pltpu.VMEM((CHUNK,), jnp.float32),
        ],
    )(functools.partial(_sc_body, n, iters))
    return f(x, max_a, agg_n, his_x, his_m, inv_w)


def kernel(x, max_a, agg_n, his_x, his_m, inv_w):
    tc_out = _tc_call(x[:N_TC], max_a[:N_TC], agg_n[:N_TC],
                      his_x[:N_TC], his_m[:N_TC], inv_w[:N_TC])
    sc_out = _sc_call(x[N_TC:], max_a[N_TC:], agg_n[N_TC:],
                      his_x[N_TC:], his_m[N_TC:], inv_w[N_TC:])
    return jnp.concatenate([tc_out, sc_out], axis=0)


# speedup vs baseline: 3.4566x; 2.8814x over previous
import jax
import jax.numpy as jnp
from jax.experimental import pallas as pl
from jax.experimental.pallas import tpu as pltpu

N, D = 100000, 128
BLOCK = 16384


def _emma_body(x_ref, max_a_ref, agg_n_ref, his_x_ref, his_m_ref, inv_w_ref,
               out_ref):
    i = pl.program_id(0)
    sl = pl.ds(i * BLOCK, BLOCK)
    max_a = max_a_ref[sl]
    his_m = his_m_ref[sl]
    beta = jnp.clip(1.0 - inv_w_ref[sl] * agg_n_ref[sl], 0.0, 1.0)
    max_m = jnp.maximum(max_a, his_m)
    neg_inf = jnp.float32(-jnp.inf)
    dp = his_m - max_m
    dq = max_a - max_m
    dp = jnp.where(jnp.isnan(dp), neg_inf, dp)
    dq = jnp.where(jnp.isnan(dq), neg_inf, dq)
    p = jnp.exp(dp) * beta
    q = jnp.exp(dq)
    t = jnp.maximum(p + q, 1.0)
    inv_t = 1.0 / t
    p2 = (p * inv_t)[:, None]
    q2 = (q * inv_t)[:, None]
    out_ref[...] = his_x_ref[...] * p2 + x_ref[...] * q2


def kernel(x, max_a, agg_n, his_x, his_m, inv_w):
    n = x.shape[0]
    n_pad = ((n + BLOCK - 1) // BLOCK) * BLOCK
    grid = n_pad // BLOCK
    row_spec = pl.BlockSpec((BLOCK, D), lambda i: (i, 0))
    vec_spec = pl.BlockSpec(memory_space=pltpu.VMEM)
    pad = n_pad - n
    max_a_p = jnp.pad(max_a, (0, pad))
    agg_n_p = jnp.pad(agg_n, (0, pad))
    his_m_p = jnp.pad(his_m, (0, pad))
    inv_w_p = jnp.pad(inv_w, (0, pad))
    return pl.pallas_call(
        _emma_body,
        grid=(grid,),
        in_specs=[row_spec, vec_spec, vec_spec, row_spec, vec_spec, vec_spec],
        out_specs=row_spec,
        out_shape=jax.ShapeDtypeStruct((n, D), jnp.float32),
        compiler_params=pltpu.CompilerParams(
            dimension_semantics=("arbitrary",),
        ),
    )(x, max_a_p, agg_n_p, his_x, his_m_p, inv_w_p)


# speedup vs baseline: 3.9543x; 1.1440x over previous
import jax
import jax.numpy as jnp
from jax.experimental import pallas as pl
from jax.experimental.pallas import tpu as pltpu

N, D = 100000, 128
BLOCK = 16384


def _emma_body(x_ref, max_a_ref, agg_n_ref, his_x_ref, his_m_ref, inv_w_ref,
               out_ref):
    max_a = max_a_ref[...]
    his_m = his_m_ref[...]
    beta = jnp.clip(1.0 - inv_w_ref[...] * agg_n_ref[...], 0.0, 1.0)
    max_m = jnp.maximum(max_a, his_m)
    neg_inf = jnp.float32(-jnp.inf)
    dp = his_m - max_m
    dq = max_a - max_m
    dp = jnp.where(jnp.isnan(dp), neg_inf, dp)
    dq = jnp.where(jnp.isnan(dq), neg_inf, dq)
    p = jnp.exp(dp) * beta
    q = jnp.exp(dq)
    t = jnp.maximum(p + q, 1.0)
    inv_t = 1.0 / t
    p2 = (p * inv_t)[:, None]
    q2 = (q * inv_t)[:, None]
    out_ref[...] = his_x_ref[...] * p2 + x_ref[...] * q2


def kernel(x, max_a, agg_n, his_x, his_m, inv_w):
    n = x.shape[0]
    grid = (n + BLOCK - 1) // BLOCK
    row_spec = pl.BlockSpec((BLOCK, D), lambda i: (i, 0))
    vec_spec = pl.BlockSpec((BLOCK,), lambda i: (i,))
    return pl.pallas_call(
        _emma_body,
        grid=(grid,),
        in_specs=[row_spec, vec_spec, vec_spec, row_spec, vec_spec, vec_spec],
        out_specs=row_spec,
        out_shape=jax.ShapeDtypeStruct((n, D), jnp.float32),
        compiler_params=pltpu.CompilerParams(
            dimension_semantics=("parallel",),
        ),
    )(x, max_a, agg_n, his_x, his_m, inv_w)


# speedup vs baseline: 4.1300x; 1.0444x over previous
import jax
import jax.numpy as jnp
from jax import lax
from jax.experimental import pallas as pl
from jax.experimental.pallas import tpu as pltpu

N, D = 100000, 128
B = 4096
FULL_STEPS = N // B
TB = N - FULL_STEPS * B
K = 4


def _scalar_math(max_a, his_m, agg_n, inv_w):
    beta = jnp.clip(1.0 - inv_w * agg_n, 0.0, 1.0)
    max_m = jnp.maximum(max_a, his_m)
    neg_inf = jnp.float32(-jnp.inf)
    dp = his_m - max_m
    dq = max_a - max_m
    dp = jnp.where(jnp.isnan(dp), neg_inf, dp)
    dq = jnp.where(jnp.isnan(dq), neg_inf, dq)
    p = jnp.exp(dp) * beta
    q = jnp.exp(dq)
    t = jnp.maximum(p + q, 1.0)
    inv_t = 1.0 / t
    return (p * inv_t)[:, None], (q * inv_t)[:, None]


def _body(x_hbm, ma_hbm, an_hbm, hm_hbm, iw_hbm, hx_hbm, out_hbm,
          xb, hb, ob, xt, ht, ot, mav, anv, hmv, iwv,
          load_sem, store_sem, scal_sem, tail_sem, tstore_sem):

    def start_load(step, slot):
        rows = pl.ds(step * B, B)
        pltpu.make_async_copy(x_hbm.at[rows], xb.at[slot], load_sem.at[slot, 0]).start()
        pltpu.make_async_copy(hx_hbm.at[rows], hb.at[slot], load_sem.at[slot, 1]).start()

    def wait_load(step, slot):
        rows = pl.ds(step * B, B)
        pltpu.make_async_copy(x_hbm.at[rows], xb.at[slot], load_sem.at[slot, 0]).wait()
        pltpu.make_async_copy(hx_hbm.at[rows], hb.at[slot], load_sem.at[slot, 1]).wait()

    pltpu.make_async_copy(ma_hbm, mav, scal_sem.at[0]).start()
    pltpu.make_async_copy(an_hbm, anv, scal_sem.at[1]).start()
    pltpu.make_async_copy(hm_hbm, hmv, scal_sem.at[2]).start()
    pltpu.make_async_copy(iw_hbm, iwv, scal_sem.at[3]).start()
    tail_rows = pl.ds(FULL_STEPS * B, TB)
    pltpu.make_async_copy(x_hbm.at[tail_rows], xt, tail_sem.at[0]).start()
    pltpu.make_async_copy(hx_hbm.at[tail_rows], ht, tail_sem.at[1]).start()
    for s in range(K - 1):
        start_load(s, s)

    pltpu.make_async_copy(ma_hbm, mav, scal_sem.at[0]).wait()
    pltpu.make_async_copy(an_hbm, anv, scal_sem.at[1]).wait()
    pltpu.make_async_copy(hm_hbm, hmv, scal_sem.at[2]).wait()
    pltpu.make_async_copy(iw_hbm, iwv, scal_sem.at[3]).wait()

    pltpu.make_async_copy(x_hbm.at[tail_rows], xt, tail_sem.at[0]).wait()
    pltpu.make_async_copy(hx_hbm.at[tail_rows], ht, tail_sem.at[1]).wait()
    tsl = pl.ds(FULL_STEPS * B, TB)
    p2, q2 = _scalar_math(mav[tsl], hmv[tsl], anv[tsl], iwv[tsl])
    ot[...] = ht[...] * p2 + xt[...] * q2
    pltpu.make_async_copy(ot, out_hbm.at[tail_rows], tstore_sem).start()

    def step_fn(i, carry):
        slot = lax.rem(i, K)

        @pl.when(i >= K)
        def _():
            rows_old = pl.ds((i - K) * B, B)
            pltpu.make_async_copy(ob.at[slot], out_hbm.at[rows_old],
                                  store_sem.at[slot]).wait()

        @pl.when(i + K - 1 < FULL_STEPS)
        def _():
            start_load(i + K - 1, lax.rem(i + K - 1, K))

        wait_load(i, slot)

        off = pl.multiple_of(i * B, B)
        sl = pl.ds(off, B)
        p2, q2 = _scalar_math(mav[sl], hmv[sl], anv[sl], iwv[sl])
        ob[slot] = hb[slot] * p2 + xb[slot] * q2

        rows = pl.ds(i * B, B)
        pltpu.make_async_copy(ob.at[slot], out_hbm.at[rows],
                              store_sem.at[slot]).start()
        return carry

    lax.fori_loop(0, FULL_STEPS, step_fn, 0)

    for i in range(FULL_STEPS - K, FULL_STEPS):
        slot = i % K
        rows = pl.ds(i * B, B)
        pltpu.make_async_copy(ob.at[slot], out_hbm.at[rows],
                              store_sem.at[slot]).wait()
    pltpu.make_async_copy(ot, out_hbm.at[tail_rows], tstore_sem).wait()


def kernel(x, max_a, agg_n, his_x, his_m, inv_w):
    any_spec = pl.BlockSpec(memory_space=pl.ANY)
    return pl.pallas_call(
        _body,
        in_specs=[any_spec] * 6,
        out_specs=any_spec,
        out_shape=jax.ShapeDtypeStruct((N, D), jnp.float32),
        scratch_shapes=[
            pltpu.VMEM((K, B, D), jnp.float32),
            pltpu.VMEM((K, B, D), jnp.float32),
            pltpu.VMEM((K, B, D), jnp.float32),
            pltpu.VMEM((TB, D), jnp.float32),
            pltpu.VMEM((TB, D), jnp.float32),
            pltpu.VMEM((TB, D), jnp.float32),
            pltpu.VMEM((N,), jnp.float32),
            pltpu.VMEM((N,), jnp.float32),
            pltpu.VMEM((N,), jnp.float32),
            pltpu.VMEM((N,), jnp.float32),
            pltpu.SemaphoreType.DMA((K, 2)),
            pltpu.SemaphoreType.DMA((K,)),
            pltpu.SemaphoreType.DMA((4,)),
            pltpu.SemaphoreType.DMA((2,)),
            pltpu.SemaphoreType.DMA,
        ],
    )(x, max_a, agg_n, his_m, inv_w, his_x)
